# baseline (speedup 1.0000x reference)
import functools

import jax
import jax.numpy as jnp
from jax import lax
from jax.experimental import pallas as pl
from jax.experimental.pallas import tpu as pltpu
from jax.experimental.pallas import tpu_sc as plsc

EMB = 128
B_TOT = 4096 * 50

_info = plsc.get_sparse_core_info()
NC = _info.num_cores
NS = _info.num_subcores
NW = NC * NS
BPW = B_TOT // NW
CH = 320
NSTEP = BPW // CH
NBUF = 2
NOUT = NSTEP // NBUF

_mesh = plsc.VectorSubcoreMesh(core_axis_name="c", subcore_axis_name="s")


@functools.partial(
    pl.kernel,
    mesh=_mesh,
    out_type=jax.ShapeDtypeStruct((B_TOT, EMB), jnp.float32),
    scratch_types=(
        [pltpu.VMEM((BPW,), jnp.int32)]
        + [pltpu.VMEM((CH, EMB), jnp.float32) for _ in range(NBUF)]
        + [pltpu.SemaphoreType.DMA for _ in range(2 * NBUF)]
    ),
)
def _embed(x_hbm, w_hbm, out_hbm, idx_v, *bufs_and_sems):
    rows = bufs_and_sems[:NBUF]
    gsem = bufs_and_sems[NBUF:2 * NBUF]
    ssem = bufs_and_sems[2 * NBUF:]

    wid = lax.axis_index("s") * NC + lax.axis_index("c")
    base = wid * BPW
    pltpu.sync_copy(x_hbm.at[pl.ds(base, BPW)], idx_v)

    def gather(g, b):
        off = pl.multiple_of(g * CH, CH)
        pltpu.async_copy(w_hbm.at[idx_v.at[pl.ds(off, CH)]], rows[b], gsem[b])

    def gather_wait(b):
        pltpu.make_async_copy(
            w_hbm.at[idx_v.at[pl.ds(0, CH)]], rows[b], gsem[b]).wait()

    def store(g, b):
        off = pl.multiple_of(g * CH, CH)
        pltpu.async_copy(rows[b], out_hbm.at[pl.ds(base + off, CH)], ssem[b])

    def store_wait(b):
        pltpu.make_async_copy(
            rows[b], out_hbm.at[pl.ds(base, CH)], ssem[b]).wait()

    for b in range(NBUF):
        gather(b, b)

    def body(it, carry):
        g0 = it * NBUF
        for b in range(NBUF):
            gather_wait(b)
            store(g0 + b, b)
        for b in range(NBUF):
            @pl.when(it < NOUT - 1)
            def _():
                store_wait(b)
                gather(g0 + NBUF + b, b)
        return carry

    lax.fori_loop(0, NOUT, body, 0)

    for b in range(NBUF):
        store_wait(b)


def kernel(x, weight):
    xf = x.reshape(-1).astype(jnp.int32)
    out = _embed(xf, weight)
    return out.reshape(x.shape + (EMB,))


# speedup vs baseline: 1.0196x; 1.0196x over previous
import functools

import jax
import jax.numpy as jnp
from jax import lax
from jax.experimental import pallas as pl
from jax.experimental.pallas import tpu as pltpu
from jax.experimental.pallas import tpu_sc as plsc

EMB = 128
B_TOT = 4096 * 50

_info = plsc.get_sparse_core_info()
NC = _info.num_cores
NS = _info.num_subcores
NW = NC * NS
BPW = B_TOT // NW
CH = 128
NSTEP = BPW // CH
NBUF = 5
NOUT = NSTEP // NBUF

_mesh = plsc.VectorSubcoreMesh(core_axis_name="c", subcore_axis_name="s")


@functools.partial(
    pl.kernel,
    mesh=_mesh,
    out_type=jax.ShapeDtypeStruct((B_TOT, EMB), jnp.float32),
    scratch_types=(
        [pltpu.VMEM((BPW,), jnp.int32)]
        + [pltpu.VMEM((CH, EMB), jnp.float32) for _ in range(NBUF)]
        + [pltpu.SemaphoreType.DMA for _ in range(2 * NBUF)]
    ),
)
def _embed(x_hbm, w_hbm, out_hbm, idx_v, *bufs_and_sems):
    rows = bufs_and_sems[:NBUF]
    gsem = bufs_and_sems[NBUF:2 * NBUF]
    ssem = bufs_and_sems[2 * NBUF:]

    wid = lax.axis_index("s") * NC + lax.axis_index("c")
    base = wid * BPW
    pltpu.sync_copy(x_hbm.at[pl.ds(base, BPW)], idx_v)

    def gather(g, b):
        off = pl.multiple_of(g * CH, CH)
        pltpu.async_copy(w_hbm.at[idx_v.at[pl.ds(off, CH)]], rows[b], gsem[b])

    def gather_wait(b):
        pltpu.make_async_copy(
            w_hbm.at[idx_v.at[pl.ds(0, CH)]], rows[b], gsem[b]).wait()

    def store(g, b):
        off = pl.multiple_of(g * CH, CH)
        pltpu.async_copy(rows[b], out_hbm.at[pl.ds(base + off, CH)], ssem[b])

    def store_wait(b):
        pltpu.make_async_copy(
            rows[b], out_hbm.at[pl.ds(base, CH)], ssem[b]).wait()

    for b in range(NBUF):
        gather(b, b)

    def body(it, carry):
        g0 = it * NBUF
        for b in range(NBUF):
            gather_wait(b)
            store(g0 + b, b)
        for b in range(NBUF):
            @pl.when(it < NOUT - 1)
            def _():
                store_wait(b)
                gather(g0 + NBUF + b, b)
        return carry

    lax.fori_loop(0, NOUT, body, 0)

    for b in range(NBUF):
        store_wait(b)


def kernel(x, weight):
    xf = x.reshape(-1).astype(jnp.int32)
    out = _embed(xf, weight)
    return out.reshape(x.shape + (EMB,))


# speedup vs baseline: 1.7588x; 1.7250x over previous
import functools

import jax
import jax.numpy as jnp
from jax import lax
from jax.experimental import pallas as pl
from jax.experimental.pallas import tpu as pltpu
from jax.experimental.pallas import tpu_sc as plsc

B = 4096
L = 50
EMB = 128

_info = plsc.get_sparse_core_info()
NC = _info.num_cores
NS = _info.num_subcores
NW = NC * NS
RPW = B // NW
NBUF = 5
NOUT = L // NBUF

_mesh = plsc.VectorSubcoreMesh(core_axis_name="c", subcore_axis_name="s")


@functools.partial(
    pl.kernel,
    mesh=_mesh,
    out_type=jax.ShapeDtypeStruct((B, L, EMB), jnp.float32),
    scratch_types=(
        [pltpu.VMEM((RPW, L), jnp.int32), pltpu.VMEM((L, RPW), jnp.int32)]
        + [pltpu.VMEM((RPW, EMB), jnp.float32) for _ in range(NBUF)]
        + [pltpu.SemaphoreType.DMA for _ in range(2 * NBUF)]
    ),
    compiler_params=pltpu.CompilerParams(use_tc_tiling_on_sc=True, needs_layout_passes=False),
)
def _embed(x_hbm, w_hbm, out_hbm, idx_raw, idx_t, *bufs_and_sems):
    rows = bufs_and_sems[:NBUF]
    gsem = bufs_and_sems[NBUF:2 * NBUF]
    ssem = bufs_and_sems[2 * NBUF:]

    wid = lax.axis_index("s") * NC + lax.axis_index("c")
    xr0 = wid * RPW
    pltpu.sync_copy(x_hbm.at[pl.ds(xr0, RPW), :], idx_raw)

    lane = lax.iota(jnp.int32, 16)

    def tbody(j, carry):
        cols = jnp.full((16,), j, jnp.int32)
        for k in range(RPW // 16):
            v = plsc.load_gather(idx_raw, [k * 16 + lane, cols])
            idx_t[j, pl.ds(k * 16, 16)] = v
        return carry

    lax.fori_loop(0, L, tbody, 0)

    def gather(j, b):
        pltpu.async_copy(w_hbm.at[idx_t.at[j]], rows[b], gsem[b])

    def gather_wait(b):
        pltpu.make_async_copy(w_hbm.at[idx_t.at[0]], rows[b], gsem[b]).wait()

    def store(j, b):
        pltpu.async_copy(rows[b], out_hbm.at[pl.ds(xr0, RPW), j, :], ssem[b])

    def store_wait(b):
        pltpu.make_async_copy(
            rows[b], out_hbm.at[pl.ds(xr0, RPW), 0, :], ssem[b]).wait()

    for b in range(NBUF):
        gather(b, b)

    def body(it, carry):
        j0 = it * NBUF
        for b in range(NBUF):
            gather_wait(b)
            store(j0 + b, b)
        for b in range(NBUF):
            @pl.when(it < NOUT - 1)
            def _():
                store_wait(b)
                gather(j0 + NBUF + b, b)
        return carry

    lax.fori_loop(0, NOUT, body, 0)

    for b in range(NBUF):
        store_wait(b)


def kernel(x, weight):
    return _embed(x, weight)


# speedup vs baseline: 1.8050x; 1.0263x over previous
import functools

import jax
import jax.numpy as jnp
from jax import lax
from jax.experimental import pallas as pl
from jax.experimental.pallas import tpu as pltpu
from jax.experimental.pallas import tpu_sc as plsc

B = 4096
L = 50
EMB = 128

_info = plsc.get_sparse_core_info()
NC = _info.num_cores
NS = _info.num_subcores
NW = NC * NS
RPW = B // NW
NBUF = 5
NOUT = L // NBUF

_mesh = plsc.VectorSubcoreMesh(core_axis_name="c", subcore_axis_name="s")


@functools.partial(
    pl.kernel,
    mesh=_mesh,
    out_type=jax.ShapeDtypeStruct((B, L, EMB), jnp.float32),
    scratch_types=(
        [pltpu.VMEM((RPW, L), jnp.int32), pltpu.VMEM((L, RPW), jnp.int32)]
        + [pltpu.VMEM((RPW, EMB), jnp.float32) for _ in range(NBUF)]
        + [pltpu.SemaphoreType.DMA for _ in range(2 * NBUF)]
    ),
    compiler_params=pltpu.CompilerParams(use_tc_tiling_on_sc=True, needs_layout_passes=False),
)
def _embed(x_hbm, w_hbm, out_hbm, idx_raw, idx_t, *bufs_and_sems):
    rows = bufs_and_sems[:NBUF]
    gsem = bufs_and_sems[NBUF:2 * NBUF]
    ssem = bufs_and_sems[2 * NBUF:]

    wid = lax.axis_index("s") * NC + lax.axis_index("c")
    xr0 = wid * RPW
    pltpu.sync_copy(x_hbm.at[pl.ds(xr0, RPW), :], idx_raw)

    lane = lax.iota(jnp.int32, 16)

    def tbody(j, carry):
        cols = jnp.full((16,), j, jnp.int32)
        for k in range(RPW // 16):
            v = plsc.load_gather(idx_raw, [k * 16 + lane, cols])
            idx_t[j, pl.ds(k * 16, 16)] = v
        return carry

    lax.fori_loop(0, NBUF, tbody, 0)

    def gather(j, b):
        pltpu.async_copy(w_hbm.at[idx_t.at[j]], rows[b], gsem[b])

    def gather_wait(b):
        pltpu.make_async_copy(w_hbm.at[idx_t.at[0]], rows[b], gsem[b]).wait()

    def store(j, b):
        pltpu.async_copy(rows[b], out_hbm.at[pl.ds(xr0, RPW), j, :], ssem[b])

    def store_wait(b):
        pltpu.make_async_copy(
            rows[b], out_hbm.at[pl.ds(xr0, RPW), 0, :], ssem[b]).wait()

    for b in range(NBUF):
        gather(b, b)

    lax.fori_loop(NBUF, L, tbody, 0)

    def body(it, carry):
        j0 = it * NBUF
        for b in range(NBUF):
            gather_wait(b)
            store(j0 + b, b)
        for b in range(NBUF):
            @pl.when(it < NOUT - 1)
            def _():
                store_wait(b)
                gather(j0 + NBUF + b, b)
        return carry

    lax.fori_loop(0, NOUT, body, 0)

    for b in range(NBUF):
        store_wait(b)


def kernel(x, weight):
    return _embed(x, weight)


# speedup vs baseline: 1.8245x; 1.0108x over previous
import functools

import jax
import jax.numpy as jnp
from jax import lax
from jax.experimental import pallas as pl
from jax.experimental.pallas import tpu as pltpu
from jax.experimental.pallas import tpu_sc as plsc

B = 4096
L = 50
EMB = 128

_info = plsc.get_sparse_core_info()
NC = _info.num_cores
NS = _info.num_subcores
NW = NC * NS
RPW = B // NW
NBUF = 8
NOUT = RPW // NBUF

_mesh = plsc.VectorSubcoreMesh(core_axis_name="c", subcore_axis_name="s")


@functools.partial(
    pl.kernel,
    mesh=_mesh,
    out_type=jax.ShapeDtypeStruct((B, L, EMB), jnp.float32),
    scratch_types=(
        [pltpu.VMEM((RPW, L), jnp.int32)]
        + [pltpu.VMEM((L, EMB), jnp.float32) for _ in range(NBUF)]
        + [pltpu.SemaphoreType.DMA for _ in range(2 * NBUF)]
    ),
    compiler_params=pltpu.CompilerParams(use_tc_tiling_on_sc=True, needs_layout_passes=False),
)
def _embed(x_hbm, w_hbm, out_hbm, idx, *bufs_and_sems):
    rows = bufs_and_sems[:NBUF]
    gsem = bufs_and_sems[NBUF:2 * NBUF]
    ssem = bufs_and_sems[2 * NBUF:]

    wid = lax.axis_index("s") * NC + lax.axis_index("c")
    xr0 = wid * RPW
    pltpu.sync_copy(x_hbm.at[pl.ds(xr0, RPW), :], idx)

    def gather(i, b):
        pltpu.async_copy(w_hbm.at[idx.at[i]], rows[b], gsem[b])

    def gather_wait(b):
        pltpu.make_async_copy(w_hbm.at[idx.at[0]], rows[b], gsem[b]).wait()

    def store(i, b):
        pltpu.async_copy(rows[b], out_hbm.at[xr0 + i], ssem[b])

    def store_wait(b):
        pltpu.make_async_copy(rows[b], out_hbm.at[xr0], ssem[b]).wait()

    for b in range(NBUF):
        gather(b, b)

    def body(it, carry):
        i0 = it * NBUF
        for b in range(NBUF):
            gather_wait(b)
            store(i0 + b, b)
        for b in range(NBUF):
            @pl.when(it < NOUT - 1)
            def _():
                store_wait(b)
                gather(i0 + NBUF + b, b)
        return carry

    lax.fori_loop(0, NOUT, body, 0)

    for b in range(NBUF):
        store_wait(b)


def kernel(x, weight):
    return _embed(x, weight)
